# Initial kernel scaffold; baseline (speedup 1.0000x reference)
import jax
import jax.numpy as jnp
from jax.experimental import pallas as pl


def kernel(a):
    raise NotImplementedError("write your pallas kernel here")


# speedup vs baseline: 2.9249x; 2.9249x over previous
import jax
import jax.numpy as jnp
from jax.experimental import pallas as pl

_K = 192
_C = 768
_HW = 1024
_B = 32
_MIN32 = -2147483648


def _body(a_ref, o_ref):
    a = a_ref[0]
    sums = jnp.sum(a, axis=1)
    sums = jnp.where(sums == 0.0, 0.0, sums).reshape(1, _C)
    bits = jax.lax.bitcast_convert_type(sums, jnp.int32)
    skey = jnp.where(bits < 0, bits ^ jnp.int32(0x7FFFFFFF), bits)

    prefix = jnp.zeros((1, 1), jnp.int32)
    for bit in range(31, -1, -1):
        cand_u = prefix | jnp.int32((1 << bit) if bit < 31 else _MIN32)
        cand_s = cand_u ^ jnp.int32(_MIN32)
        cnt = jnp.sum(jnp.where(skey >= cand_s, 1, 0).astype(jnp.int32),
                      axis=1, keepdims=True)
        prefix = jnp.where(cnt >= _K, cand_u, prefix)
    t_s = prefix ^ jnp.int32(_MIN32)

    mask_gt = skey > t_s
    cnt_gt = jnp.sum(jnp.where(mask_gt, 1, 0).astype(jnp.int32),
                     axis=1, keepdims=True)
    need = _K - cnt_gt
    eq = skey == t_s
    idxv = jax.lax.broadcasted_iota(jnp.int32, (1, _C), 1)
    r = jnp.zeros((1, 1), jnp.int32)
    for bit in range(9, -1, -1):
        cand = r | jnp.int32((1 << bit) - 1)
        cnt = jnp.sum(jnp.where(eq & (idxv <= cand), 1, 0).astype(jnp.int32),
                      axis=1, keepdims=True)
        r = jnp.where(cnt >= need, r, r | jnp.int32(1 << bit))
    mask_tie = eq & (idxv <= r) & (need > 0)

    mask = jnp.where(mask_gt | mask_tie, 1.0, 0.0).astype(jnp.float32)
    out = jax.lax.dot(mask, a, preferred_element_type=jnp.float32,
                      precision=jax.lax.Precision.HIGHEST)
    o_ref[0] = out * jnp.float32(1.0 / _K)


def kernel(a):
    a3 = a.reshape(_B, _C, _HW)
    out = pl.pallas_call(
        _body,
        grid=(_B,),
        in_specs=[pl.BlockSpec((1, _C, _HW), lambda b: (b, 0, 0))],
        out_specs=pl.BlockSpec((1, 1, _HW), lambda b: (b, 0, 0)),
        out_shape=jax.ShapeDtypeStruct((_B, 1, _HW), jnp.float32),
    )(a3)
    return out.reshape(_B, 1, 32, 32)


# speedup vs baseline: 9.9558x; 3.4038x over previous
import jax
import jax.numpy as jnp
from jax.experimental import pallas as pl

_K = 192
_C = 768
_HW = 1024
_B = 32
_MIN32 = -2147483648


def _body(a_ref, o_ref):
    a = a_ref[0]
    sums = jnp.sum(a, axis=1)
    sums = jnp.where(sums == 0.0, 0.0, sums)
    bits = jax.lax.bitcast_convert_type(sums, jnp.int32)
    skey = jnp.where(bits < 0, bits ^ jnp.int32(0x7FFFFFFF), bits)

    krow = skey.reshape(1, _C)
    kcol = skey.reshape(_C, 1)
    irow = jax.lax.broadcasted_iota(jnp.int32, (1, _C), 1)
    icol = jax.lax.broadcasted_iota(jnp.int32, (_C, 1), 0)
    beats = (kcol > krow) | ((kcol == krow) & (icol < irow))
    rank = jnp.sum(jnp.where(beats, 1, 0).astype(jnp.int32),
                   axis=0, keepdims=True)
    mask = jnp.where(rank < _K, 1.0, 0.0).astype(jnp.float32)
    out = jax.lax.dot(mask, a, preferred_element_type=jnp.float32,
                      precision=jax.lax.Precision.HIGHEST)
    o_ref[0] = out * jnp.float32(1.0 / _K)


def kernel(a):
    a3 = a.reshape(_B, _C, _HW)
    out = pl.pallas_call(
        _body,
        grid=(_B,),
        in_specs=[pl.BlockSpec((1, _C, _HW), lambda b: (b, 0, 0))],
        out_specs=pl.BlockSpec((1, 1, _HW), lambda b: (b, 0, 0)),
        out_shape=jax.ShapeDtypeStruct((_B, 1, _HW), jnp.float32),
    )(a3)
    return out.reshape(_B, 1, 32, 32)


# speedup vs baseline: 11.3597x; 1.1410x over previous
import jax
import jax.numpy as jnp
from jax.experimental import pallas as pl

_K = 192
_C = 768
_HW = 1024
_B = 32
_MIN32 = -2147483648


def _body(a_ref, o_ref):
    a = a_ref[0]
    sums = jnp.sum(a, axis=1)
    sums = jnp.where(sums == 0.0, 0.0, sums)
    bits = jax.lax.bitcast_convert_type(sums, jnp.int32)
    skey = jnp.where(bits < 0, bits ^ jnp.int32(0x7FFFFFFF), bits)

    krow = skey.reshape(1, _C)
    kcol = skey.reshape(_C, 1)
    irow = jax.lax.broadcasted_iota(jnp.int32, (1, _C), 1)
    icol = jax.lax.broadcasted_iota(jnp.int32, (_C, 1), 0)
    beats = (kcol > krow) | ((kcol == krow) & (icol < irow))
    rank = jnp.sum(jnp.where(beats, 1, 0).astype(jnp.int32),
                   axis=0, keepdims=True)
    mask = jnp.where(rank < _K, 1.0, 0.0).astype(jnp.float32)
    out = jax.lax.dot(mask, a, preferred_element_type=jnp.float32)
    o_ref[0] = out * jnp.float32(1.0 / _K)


def kernel(a):
    a3 = a.reshape(_B, _C, _HW)
    out = pl.pallas_call(
        _body,
        grid=(_B,),
        in_specs=[pl.BlockSpec((1, _C, _HW), lambda b: (b, 0, 0))],
        out_specs=pl.BlockSpec((1, 1, _HW), lambda b: (b, 0, 0)),
        out_shape=jax.ShapeDtypeStruct((_B, 1, _HW), jnp.float32),
    )(a3)
    return out.reshape(_B, 1, 32, 32)
